# baseline (speedup 1.0000x reference)
import jax
import jax.numpy as jnp
from jax.experimental import pallas as pl
from jax.experimental.pallas import tpu as pltpu

_F8 = jnp.float8_e4m3fn
_CACHE_BYTES = 23 * 1024 * 1024


def _pick_bm(n: int, target: int) -> int:
    best = None
    for bm in range(8, target + 1, 8):
        if n % bm == 0:
            best = bm
    return best if best is not None else n


def _s1_body(x_ref, w1_ref, s1_ref):
    xb = x_ref[...].astype(jnp.bfloat16)
    wb = w1_ref[...].astype(jnp.bfloat16)
    s1_ref[...] = jnp.dot(xb, wb, preferred_element_type=jnp.float32).astype(
        jnp.bfloat16
    )


def _make_fused_body(bm, n, nm, nc, nout):
    nsp = nm - nc

    def body(adj_ref, s1_ref, b1_ref, w2_ref, b2_ref, out_ref, spill_ref,
             cache_ref, stage0, stage1, s2f_ref, s2q_ref,
             sem_w0, sem_w1, sem_r0, sem_r1):
        i = pl.program_id(0)
        stages = (stage0, stage1)
        sems_w = (sem_w0, sem_w1)
        sems_r = (sem_r0, sem_r1)

        def spill_at(blk):
            return spill_ref.at[pl.ds(blk * bm, bm), :]

        @pl.when(i < nm)
        def _phase1():
            a = adj_ref[...]
            a8 = a.astype(_F8)

            @pl.when(i < nc)
            def _():
                cache_ref[i] = a8

            if nsp > 0:
                for par in (0, 1):
                    @pl.when((i >= nc) & (i % 2 == par))
                    def _(par=par):
                        stg, sem = stages[par], sems_w[par]

                        @pl.when(i >= nc + 2)
                        def _():
                            pltpu.make_async_copy(
                                stg, spill_at(i - 2 - nc), sem
                            ).wait()

                        stg[...] = a8
                        pltpu.make_async_copy(
                            stg, spill_at(i - nc), sem
                        ).start()

            acc = jnp.dot(
                a.astype(jnp.bfloat16), s1_ref[...],
                preferred_element_type=jnp.float32,
            )
            h = jnp.maximum(acc + b1_ref[...], 0.0)
            s2b = jnp.dot(
                h.astype(jnp.bfloat16), w2_ref[...].astype(jnp.bfloat16),
                preferred_element_type=jnp.float32,
            )
            s2f_ref[pl.ds(i * bm, bm), :] = s2b

        @pl.when(i >= nm)
        def _phase2():
            j = i - nm

            @pl.when(j == 0)
            def _():
                s2q_ref[...] = s2f_ref[...].astype(_F8)

            s2_all = s2q_ref[...]

            if nsp > 0:
                @pl.when(j == 0)
                def _():
                    for par in (0, 1):
                        last = nsp - 2 + ((nsp + par) % 2)
                        pltpu.make_async_copy(
                            stages[par], spill_at(last), sems_w[par]
                        ).wait()

                j2 = j + 1
                for par in (0, 1):
                    @pl.when((j2 >= nc) & (j2 < nm) & (j2 % 2 == par))
                    def _(par=par):
                        pltpu.make_async_copy(
                            spill_at(j2 - nc), stages[par], sems_r[par]
                        ).start()

            @pl.when(j < nc)
            def _():
                a8 = cache_ref[jnp.minimum(j, nc - 1)]
                out_ref[...] = jnp.dot(
                    a8, s2_all, preferred_element_type=jnp.float32
                ) + b2_ref[...]

            if nsp > 0:
                for par in (0, 1):
                    @pl.when((j >= nc) & (j % 2 == par))
                    def _(par=par):
                        pltpu.make_async_copy(
                            spill_at(j - nc), stages[par], sems_r[par]
                        ).wait()
                        out_ref[...] = jnp.dot(
                            stages[par][...], s2_all,
                            preferred_element_type=jnp.float32,
                        ) + b2_ref[...]

    return body


def kernel(x, adj, W1, b1, W2, b2):
    n, _ = x.shape
    nhid = W1.shape[1]
    nout = W2.shape[1]
    bm = _pick_bm(n, 200)
    nm = n // bm
    plane_bytes = (-(-bm // 32) * 32) * (-(-n // 128) * 128)
    nc = max(1, min(nm, _CACHE_BYTES // plane_bytes))
    if nm - nc == 1:
        nc -= 1
    nsp = nm - nc

    s1 = pl.pallas_call(
        _s1_body,
        out_shape=jax.ShapeDtypeStruct((n, nhid), jnp.bfloat16),
    )(x, W1)

    b1r = b1.reshape(1, nhid)
    b2r = b2.reshape(1, nout)

    out, _spill = pl.pallas_call(
        _make_fused_body(bm, n, nm, nc, nout),
        grid=(2 * nm,),
        in_specs=[
            pl.BlockSpec((bm, n), lambda i: (jnp.minimum(i, nm - 1), 0)),
            pl.BlockSpec((n, nhid), lambda i: (0, 0)),
            pl.BlockSpec((1, nhid), lambda i: (0, 0)),
            pl.BlockSpec((nhid, nout), lambda i: (0, 0)),
            pl.BlockSpec((1, nout), lambda i: (0, 0)),
        ],
        out_specs=[
            pl.BlockSpec((bm, nout), lambda i: (jnp.maximum(i - nm, 0), 0)),
            pl.BlockSpec(memory_space=pltpu.MemorySpace.HBM),
        ],
        out_shape=[
            jax.ShapeDtypeStruct((n, nout), jnp.float32),
            jax.ShapeDtypeStruct((max(nsp, 1) * bm, n), _F8),
        ],
        scratch_shapes=[
            pltpu.VMEM((max(nc, 1), bm, n), _F8),
            pltpu.VMEM((bm, n), _F8),
            pltpu.VMEM((bm, n), _F8),
            pltpu.VMEM((n, nhid), jnp.float32),
            pltpu.VMEM((n, nhid), _F8),
            pltpu.SemaphoreType.DMA,
            pltpu.SemaphoreType.DMA,
            pltpu.SemaphoreType.DMA,
            pltpu.SemaphoreType.DMA,
        ],
    )(adj, s1, b1r, W2, b2r)

    return out


# speedup vs baseline: 1.1829x; 1.1829x over previous
import jax
import jax.numpy as jnp
from jax.experimental import pallas as pl
from jax.experimental.pallas import tpu as pltpu

_F8 = jnp.float8_e4m3fn


def _pick_bm(n: int, target: int = 400) -> int:
    best = None
    for bm in range(8, target + 1, 8):
        if n % bm == 0:
            best = bm
    return best if best is not None else n


def _gc1_body(adj_ref, x_ref, w1_ref, b1_ref, w2_ref, s2_ref, adj8_ref,
              s1_ref):
    i = pl.program_id(0)

    @pl.when(i == 0)
    def _():
        xb = x_ref[...].astype(jnp.bfloat16)
        wb = w1_ref[...].astype(jnp.bfloat16)
        s1_ref[...] = jnp.dot(
            xb, wb, preferred_element_type=jnp.float32
        ).astype(jnp.bfloat16)

    a = adj_ref[...]
    adj8_ref[...] = a.astype(_F8)
    acc = jnp.dot(
        a.astype(jnp.bfloat16), s1_ref[...], preferred_element_type=jnp.float32
    )
    h = jnp.maximum(acc + b1_ref[...], 0.0)
    w2b = w2_ref[...].astype(jnp.bfloat16)
    s2 = jnp.dot(h.astype(jnp.bfloat16), w2b, preferred_element_type=jnp.float32)
    s2_ref[...] = s2.astype(_F8)


def _gc2_body(adj8_ref, s2_ref, b2_ref, out_ref):
    acc = jnp.dot(
        adj8_ref[...], s2_ref[...], preferred_element_type=jnp.float32
    )
    out_ref[...] = acc + b2_ref[...]


def kernel(x, adj, W1, b1, W2, b2):
    n, _ = x.shape
    nhid = W1.shape[1]
    nout = W2.shape[1]
    bm = _pick_bm(n)
    nm = n // bm

    b1r = b1.reshape(1, nhid)
    b2r = b2.reshape(1, nout)

    s2, adj8 = pl.pallas_call(
        _gc1_body,
        grid=(nm,),
        in_specs=[
            pl.BlockSpec((bm, n), lambda i: (i, 0)),
            pl.BlockSpec((n, W1.shape[0]), lambda i: (0, 0)),
            pl.BlockSpec(W1.shape, lambda i: (0, 0)),
            pl.BlockSpec((1, nhid), lambda i: (0, 0)),
            pl.BlockSpec((nhid, nout), lambda i: (0, 0)),
        ],
        out_specs=[
            pl.BlockSpec((bm, nout), lambda i: (i, 0)),
            pl.BlockSpec((bm, n), lambda i: (i, 0)),
        ],
        out_shape=[
            jax.ShapeDtypeStruct((n, nout), _F8),
            jax.ShapeDtypeStruct((n, n), _F8),
        ],
        scratch_shapes=[
            pltpu.VMEM((n, nhid), jnp.bfloat16),
        ],
    )(adj, x, W1, b1r, W2)

    out = pl.pallas_call(
        _gc2_body,
        grid=(nm,),
        in_specs=[
            pl.BlockSpec((bm, n), lambda i: (i, 0)),
            pl.BlockSpec((n, nout), lambda i: (0, 0)),
            pl.BlockSpec((1, nout), lambda i: (0, 0)),
        ],
        out_specs=pl.BlockSpec((bm, nout), lambda i: (i, 0)),
        out_shape=jax.ShapeDtypeStruct((n, nout), jnp.float32),
    )(adj8, s2, b2r)

    return out


# speedup vs baseline: 1.2600x; 1.0652x over previous
import jax
import jax.numpy as jnp
from jax.experimental import pallas as pl
from jax.experimental.pallas import tpu as pltpu

_F8 = jnp.float8_e4m3fn


def _pick_bm(n: int, target: int = 400) -> int:
    best = None
    for bm in range(8, target + 1, 8):
        if n % bm == 0:
            best = bm
    return best if best is not None else n


def _gc1_body(adj_ref, x_ref, w1_ref, b1_ref, w2_ref, s2_ref, adj8_ref,
              s1_ref):
    i = pl.program_id(0)

    @pl.when(i == 0)
    def _():
        xb = x_ref[...].astype(jnp.bfloat16)
        wb = w1_ref[...].astype(jnp.bfloat16)
        s1_ref[...] = jnp.dot(
            xb, wb, preferred_element_type=jnp.float32
        ).astype(jnp.bfloat16)

    a = adj_ref[...]
    adj8_ref[...] = a.astype(_F8)
    acc = jnp.dot(
        a.astype(jnp.bfloat16), s1_ref[...], preferred_element_type=jnp.float32
    )
    h = jnp.maximum(acc + b1_ref[...], 0.0)
    w2b = w2_ref[...].astype(jnp.bfloat16)
    s2 = jnp.dot(h.astype(jnp.bfloat16), w2b, preferred_element_type=jnp.float32)
    s2_ref[...] = s2.astype(_F8)


def _gc2_body(adj8_ref, s2_ref, b2_ref, out_ref):
    acc = jnp.dot(
        adj8_ref[...], s2_ref[...], preferred_element_type=jnp.float32
    )
    out_ref[...] = acc + b2_ref[...]


def kernel(x, adj, W1, b1, W2, b2):
    n, _ = x.shape
    nhid = W1.shape[1]
    nout = W2.shape[1]
    bm = _pick_bm(n)
    nm = n // bm

    b1r = b1.reshape(1, nhid)
    b2r = b2.reshape(1, nout)

    s2, adj8 = pl.pallas_call(
        _gc1_body,
        grid=(nm,),
        in_specs=[
            pl.BlockSpec((bm, n), lambda i: (i, 0)),
            pl.BlockSpec((n, W1.shape[0]), lambda i: (0, 0)),
            pl.BlockSpec(W1.shape, lambda i: (0, 0)),
            pl.BlockSpec((1, nhid), lambda i: (0, 0)),
            pl.BlockSpec((nhid, nout), lambda i: (0, 0)),
        ],
        out_specs=[
            pl.BlockSpec((bm, nout), lambda i: (i, 0)),
            pl.BlockSpec((bm, n), lambda i: (i, 0)),
        ],
        out_shape=[
            jax.ShapeDtypeStruct((n, nout), _F8),
            jax.ShapeDtypeStruct((n, n), _F8),
        ],
        scratch_shapes=[
            pltpu.VMEM((n, nhid), jnp.bfloat16),
        ],
    )(adj, x, W1, b1r, W2)

    bm2 = _pick_bm(n, 1000)
    nm2 = n // bm2
    out = pl.pallas_call(
        _gc2_body,
        grid=(nm2,),
        in_specs=[
            pl.BlockSpec((bm2, n), lambda i: (i, 0)),
            pl.BlockSpec((n, nout), lambda i: (0, 0)),
            pl.BlockSpec((1, nout), lambda i: (0, 0)),
        ],
        out_specs=pl.BlockSpec((bm2, nout), lambda i: (i, 0)),
        out_shape=jax.ShapeDtypeStruct((n, nout), jnp.float32),
    )(adj8, s2, b2r)

    return out
